# baseline (speedup 1.0000x reference)
import jax
import jax.numpy as jnp
from jax import lax
from jax.experimental import pallas as pl
from jax.experimental.pallas import tpu as pltpu
from jax.experimental.pallas import tpu_sc as plsc

D = 28
L = 16
NC, NS = 2, 16
NW = NC * NS
B = 128
RB = 750

_params = pltpu.CompilerParams(use_tc_tiling_on_sc=False,
                               needs_layout_passes=False)


def _worker_id():
    return lax.axis_index("s") * NC + lax.axis_index("c")


def _gather_start(src_hbm, idx_ref, dst, sem):
    return pltpu.async_copy(src_hbm.at[idx_ref], dst, sem)


def _pad_body(data_hbm, out_hbm, buf28, buf32):
    nrows = data_hbm.shape[0]
    rpw = nrows // NW
    rb = min(RB, rpw)
    rstart = _worker_id() * rpw

    def blk(g, carry):
        base = rstart + g * rb
        pltpu.sync_copy(data_hbm.at[pl.ds(base, rb)], buf28)

        def row(r, c):
            buf32[r, pl.ds(0, L)] = buf28[r, pl.ds(0, L)]
            buf32[r, pl.ds(D - L, L)] = buf28[r, pl.ds(D - L, L)]
            return c

        lax.fori_loop(0, rb, row, 0)
        pltpu.sync_copy(buf32, out_hbm.at[pl.ds(base, rb)])
        return carry

    lax.fori_loop(0, rpw // rb, blk, 0)


def _body(idx_hbm, nids_hbm, data_hbm, w_hbm, out_hbm,
          idx_v, cid_v, cix_v, w_v, rows_v, out_v, sem_n, sem_d):
    n = idx_hbm.shape[0]
    qpw = n // NW
    nblk = qpw // B
    wstart = _worker_id() * qpw
    nch = (B * 8) // 128
    io = lax.iota(jnp.int32, L)
    pat_q = lax.shift_right_logical(io, 3)
    pat_j = lax.bitwise_and(io, 7)

    def step(g, carry):
        qbase = wstart + g * B
        pltpu.sync_copy(idx_hbm.at[pl.ds(qbase, B)], idx_v)
        pltpu.sync_copy(w_hbm.at[pl.ds(qbase, B)], w_v)
        _gather_start(nids_hbm, idx_v, cid_v, sem_n).wait()
        for k in range(B * 8 // L):
            v = plsc.load_gather(cid_v, [pat_q + 2 * k, pat_j])
            cix_v[k // 8, pl.ds((k % 8) * L, L)] = v
        gathers = [
            _gather_start(data_hbm, cix_v.at[c],
                          rows_v.at[pl.ds(c * 128, 128), :], sem_d)
            for c in range(nch)
        ]
        for gth in gathers:
            gth.wait()

        def qstep(p, c):
            w_pair = plsc.load_gather(w_v, [pat_q + 2 * p, pat_j])
            for h in range(2):
                q = 2 * p + h
                acc0 = jnp.zeros((L,), jnp.float32)
                acc1 = jnp.zeros((L,), jnp.float32)
                for j in range(8):
                    wj = w_pair.at[jnp.full((L,), 8 * h + j, jnp.int32)].get(
                        mode="promise_in_bounds")
                    r0 = rows_v[8 * q + j, pl.ds(0, L)]
                    r1 = rows_v[8 * q + j, pl.ds(D - L, L)]
                    acc0 = acc0 + wj * r0
                    acc1 = acc1 + wj * r1
                out_v[q, pl.ds(0, L)] = acc0
                out_v[q, pl.ds(D - L, L)] = acc1
            return c

        lax.fori_loop(0, B // 2, qstep, 0)
        pltpu.sync_copy(out_v, out_hbm.at[pl.ds(qbase, B)])
        return carry

    lax.fori_loop(0, nblk, step, 0)


def kernel(indices, nids, data, weights):
    n = indices.shape[0]
    ncorners = data.shape[0]
    mesh = plsc.VectorSubcoreMesh(core_axis_name="c", subcore_axis_name="s",
                                  num_cores=NC, num_subcores=NS)
    rb = min(RB, ncorners // NW)
    pad_f = pl.kernel(
        _pad_body,
        out_type=jax.ShapeDtypeStruct((ncorners, 32), jnp.float32),
        mesh=mesh,
        compiler_params=_params,
        scratch_types=[
            pltpu.VMEM((rb, D), jnp.float32),
            pltpu.VMEM((rb, 32), jnp.float32),
        ],
    )
    data32 = pad_f(data)
    f = pl.kernel(
        _body,
        out_type=jax.ShapeDtypeStruct((n, D), jnp.float32),
        mesh=mesh,
        compiler_params=_params,
        scratch_types=[
            pltpu.VMEM((B,), jnp.int32),
            pltpu.VMEM((B, 8), jnp.int32),
            pltpu.VMEM((B * 8 // 128, 128), jnp.int32),
            pltpu.VMEM((B, 8), jnp.float32),
            pltpu.VMEM((B * 8, 32), jnp.float32),
            pltpu.VMEM((B, D), jnp.float32),
            pltpu.SemaphoreType.DMA,
            pltpu.SemaphoreType.DMA,
        ],
    )
    return f(indices, nids, data32, weights)


# speedup vs baseline: 1.0359x; 1.0359x over previous
import jax
import jax.numpy as jnp
from jax import lax
from jax.experimental import pallas as pl
from jax.experimental.pallas import tpu as pltpu
from jax.experimental.pallas import tpu_sc as plsc

D = 28
L = 16
NC, NS = 2, 16
NW = NC * NS
B = 128
RB = 750

_params = pltpu.CompilerParams(use_tc_tiling_on_sc=False,
                               needs_layout_passes=False)


def _worker_id():
    return lax.axis_index("s") * NC + lax.axis_index("c")


def _gather_start(src_hbm, idx_ref, dst, sem):
    return pltpu.async_copy(src_hbm.at[idx_ref], dst, sem)


def _pad_body(data_hbm, out_hbm, buf28, buf32):
    nrows = data_hbm.shape[0]
    rpw = nrows // NW
    rb = min(RB, rpw)
    rstart = _worker_id() * rpw

    def blk(g, carry):
        base = rstart + g * rb
        pltpu.sync_copy(data_hbm.at[pl.ds(base, rb)], buf28)

        def row(r, c):
            buf32[r, pl.ds(0, L)] = buf28[r, pl.ds(0, L)]
            buf32[r, pl.ds(D - L, L)] = buf28[r, pl.ds(D - L, L)]
            return c

        lax.fori_loop(0, rb, row, 0)
        pltpu.sync_copy(buf32, out_hbm.at[pl.ds(base, rb)])
        return carry

    lax.fori_loop(0, rpw // rb, blk, 0)


def _body(idx_hbm, nids_hbm, data_hbm, w_hbm, out_hbm,
          idx_v, cid_v, cix_v, w_v, rows_v, out_v, sem_n, sem_d):
    n = idx_hbm.shape[0]
    qpw = n // NW
    nblk = qpw // B
    wstart = _worker_id() * qpw
    nch = (B * 8) // 128
    io = lax.iota(jnp.int32, L)
    pat_q = lax.shift_right_logical(io, 3)
    pat_j = lax.bitwise_and(io, 7)

    def step(g, carry):
        qbase = wstart + g * B
        pltpu.sync_copy(idx_hbm.at[pl.ds(qbase, B)], idx_v)
        pltpu.sync_copy(w_hbm.at[pl.ds(qbase, B)], w_v)
        _gather_start(nids_hbm, idx_v, cid_v, sem_n).wait()
        for k in range(B * 8 // L):
            v = plsc.load_gather(cid_v, [pat_q + 2 * k, pat_j])
            cix_v[k // 8, pl.ds((k % 8) * L, L)] = v
        gathers = [
            _gather_start(data_hbm, cix_v.at[c],
                          rows_v.at[pl.ds(c * 128, 128), :], sem_d)
            for c in range(nch)
        ]
        for gth in gathers:
            gth.wait()

        def qstep(p, c):
            w_pair = plsc.load_gather(w_v, [pat_q + 2 * p, pat_j])
            for h in range(2):
                q = 2 * p + h
                acc0 = jnp.zeros((L,), jnp.float32)
                acc1 = jnp.zeros((L,), jnp.float32)
                for j in range(8):
                    wj = w_pair.at[jnp.full((L,), 8 * h + j, jnp.int32)].get(
                        mode="promise_in_bounds")
                    r0 = rows_v[8 * q + j, pl.ds(0, L)]
                    r1 = rows_v[8 * q + j, pl.ds(D - L, L)]
                    acc0 = acc0 + wj * r0
                    acc1 = acc1 + wj * r1
                out_v[q, pl.ds(0, L)] = acc0
                out_v[q, pl.ds(D - L, L)] = acc1
            return c

        lax.fori_loop(0, B // 2, qstep, 0)
        pltpu.sync_copy(out_v, out_hbm.at[pl.ds(qbase, B)])
        return carry

    lax.fori_loop(0, nblk, step, 0)


def kernel(indices, nids, data, weights):
    n = indices.shape[0]
    mesh = plsc.VectorSubcoreMesh(core_axis_name="c", subcore_axis_name="s",
                                  num_cores=NC, num_subcores=NS)
    data32 = jnp.pad(data, ((0, 0), (0, 32 - D)))
    f = pl.kernel(
        _body,
        out_type=jax.ShapeDtypeStruct((n, D), jnp.float32),
        mesh=mesh,
        compiler_params=_params,
        scratch_types=[
            pltpu.VMEM((B,), jnp.int32),
            pltpu.VMEM((B, 8), jnp.int32),
            pltpu.VMEM((B * 8 // 128, 128), jnp.int32),
            pltpu.VMEM((B, 8), jnp.float32),
            pltpu.VMEM((B * 8, 32), jnp.float32),
            pltpu.VMEM((B, D), jnp.float32),
            pltpu.SemaphoreType.DMA,
            pltpu.SemaphoreType.DMA,
        ],
    )
    return f(indices, nids, data32, weights)


# speedup vs baseline: 1.0496x; 1.0132x over previous
import jax
import jax.numpy as jnp
from jax import lax
from jax.experimental import pallas as pl
from jax.experimental.pallas import tpu as pltpu
from jax.experimental.pallas import tpu_sc as plsc

D = 28
L = 16
NC, NS = 2, 16
NW = NC * NS
B = 128
CT = 2048

_params = pltpu.CompilerParams(use_tc_tiling_on_sc=False,
                               needs_layout_passes=False)


def _worker_id():
    return lax.axis_index("s") * NC + lax.axis_index("c")


def _gather_start(src_hbm, idx_ref, dst, sem):
    return pltpu.async_copy(src_hbm.at[idx_ref], dst, sem)


def _transpose_body(nt_hbm, out_hbm, in_v, out_v):
    ncols = nt_hbm.shape[1]
    cpw = ncols // NW
    cstart = _worker_id() * cpw
    io = lax.iota(jnp.int32, L)

    def blk(g, carry):
        base = cstart + g * CT
        for j in range(8):
            pltpu.sync_copy(nt_hbm.at[j, pl.ds(base, CT)], in_v.at[j])

        def grp(t, c):
            o = t * L
            for j in range(8):
                v = in_v[j, pl.ds(o, L)]
                plsc.store_scatter(out_v, [o + io, jnp.full((L,), j, jnp.int32)], v)
            return c

        lax.fori_loop(0, CT // L, grp, 0)
        pltpu.sync_copy(out_v, out_hbm.at[pl.ds(base, CT)])
        return carry

    lax.fori_loop(0, cpw // CT, blk, 0)


def _body(idx_hbm, nids_hbm, data_hbm, wt_hbm, out_hbm,
          idx_v, cid_v, cix_v, w_v, rows_v, out_v, sem_n, sem_d):
    n = idx_hbm.shape[0]
    qpw = n // NW
    nblk = qpw // B
    wstart = _worker_id() * qpw
    nch = (B * 8) // 128
    io = lax.iota(jnp.int32, L)
    pat_q = lax.shift_right_logical(io, 3)
    pat_j = lax.bitwise_and(io, 7)

    def step(g, carry):
        qbase = wstart + g * B
        pltpu.sync_copy(idx_hbm.at[pl.ds(qbase, B)], idx_v)
        for j in range(8):
            pltpu.sync_copy(wt_hbm.at[j, pl.ds(qbase, B)], w_v.at[j])
        _gather_start(nids_hbm, idx_v, cid_v, sem_n).wait()
        for k in range(B * 8 // L):
            v = plsc.load_gather(cid_v, [pat_q + 2 * k, pat_j])
            cix_v[k // 8, pl.ds((k % 8) * L, L)] = v
        gathers = [
            _gather_start(data_hbm, cix_v.at[c],
                          rows_v.at[pl.ds(c * 128, 128), :], sem_d)
            for c in range(nch)
        ]
        for gth in gathers:
            gth.wait()

        def qstep(p, c):
            w_pair = plsc.load_gather(w_v, [pat_j, pat_q + 2 * p])
            for h in range(2):
                q = 2 * p + h
                acc0 = jnp.zeros((L,), jnp.float32)
                acc1 = jnp.zeros((L,), jnp.float32)
                for j in range(8):
                    wj = w_pair.at[jnp.full((L,), 8 * h + j, jnp.int32)].get(
                        mode="promise_in_bounds")
                    r0 = rows_v[8 * q + j, pl.ds(0, L)]
                    r1 = rows_v[8 * q + j, pl.ds(D - L, L)]
                    acc0 = acc0 + wj * r0
                    acc1 = acc1 + wj * r1
                out_v[q, pl.ds(0, L)] = acc0
                out_v[q, pl.ds(D - L, L)] = acc1
            return c

        lax.fori_loop(0, B // 2, qstep, 0)
        pltpu.sync_copy(out_v, out_hbm.at[pl.ds(qbase, B)])
        return carry

    lax.fori_loop(0, nblk, step, 0)


def kernel(indices, nids, data, weights):
    n = indices.shape[0]
    nnodes = nids.shape[0]
    mesh = plsc.VectorSubcoreMesh(core_axis_name="c", subcore_axis_name="s",
                                  num_cores=NC, num_subcores=NS)
    data32 = jnp.pad(data, ((0, 0), (0, 32 - D)))
    tr_f = pl.kernel(
        _transpose_body,
        out_type=jax.ShapeDtypeStruct((nnodes, 8), jnp.int32),
        mesh=mesh,
        compiler_params=_params,
        scratch_types=[
            pltpu.VMEM((8, CT), jnp.int32),
            pltpu.VMEM((CT, 8), jnp.int32),
        ],
    )
    nids_rm = tr_f(nids.T)
    f = pl.kernel(
        _body,
        out_type=jax.ShapeDtypeStruct((n, D), jnp.float32),
        mesh=mesh,
        compiler_params=_params,
        scratch_types=[
            pltpu.VMEM((B,), jnp.int32),
            pltpu.VMEM((B, 8), jnp.int32),
            pltpu.VMEM((B * 8 // 128, 128), jnp.int32),
            pltpu.VMEM((8, B), jnp.float32),
            pltpu.VMEM((B * 8, 32), jnp.float32),
            pltpu.VMEM((B, D), jnp.float32),
            pltpu.SemaphoreType.DMA,
            pltpu.SemaphoreType.DMA,
        ],
    )
    return f(indices, nids_rm, data32, weights.T)


# speedup vs baseline: 1.1475x; 1.0933x over previous
import jax
import jax.numpy as jnp
from jax import lax
from jax.experimental import pallas as pl
from jax.experimental.pallas import tpu as pltpu
from jax.experimental.pallas import tpu_sc as plsc

D = 28
L = 16
NC, NS = 2, 16
NW = NC * NS
B = 128
CT = 2048

_params = pltpu.CompilerParams(use_tc_tiling_on_sc=False,
                               needs_layout_passes=False)


def _worker_id():
    return lax.axis_index("s") * NC + lax.axis_index("c")


def _gather_start(src_hbm, idx_ref, dst, sem):
    return pltpu.async_copy(src_hbm.at[idx_ref], dst, sem)


def _transpose_body(nt_hbm, out_hbm, in_v, out_v):
    ncols = nt_hbm.shape[1]
    cpw = ncols // NW
    cstart = _worker_id() * cpw
    io = lax.iota(jnp.int32, L)

    def blk(g, carry):
        base = cstart + g * CT
        for j in range(8):
            pltpu.sync_copy(nt_hbm.at[j, pl.ds(base, CT)], in_v.at[j])

        def grp(t, c):
            o = t * L
            for j in range(8):
                v = in_v[j, pl.ds(o, L)]
                plsc.store_scatter(out_v, [o + io, jnp.full((L,), j, jnp.int32)], v)
            return c

        lax.fori_loop(0, CT // L, grp, 0)
        pltpu.sync_copy(out_v, out_hbm.at[pl.ds(base, CT)])
        return carry

    lax.fori_loop(0, cpw // CT, blk, 0)


def _drain(data_hbm, rows_v, sem):
    pltpu.make_async_copy(data_hbm.at[pl.ds(0, rows_v.shape[0])],
                          rows_v, sem).wait()


def _body(idx_hbm, nids_hbm, data_hbm, wt_hbm, out_hbm,
          idx_v, cid_v, cix0, cix1, w_v, rows0, rows1, out_v, sem_n, sem_d):
    n = idx_hbm.shape[0]
    qpw = n // NW
    nblk = qpw // B
    wstart = _worker_id() * qpw
    nch = (B * 8) // 128
    io = lax.iota(jnp.int32, L)
    pat_q = lax.shift_right_logical(io, 3)
    pat_j = lax.bitwise_and(io, 7)

    def stage1(qbase, cix_v, rows_v):
        pltpu.sync_copy(idx_hbm.at[pl.ds(qbase, B)], idx_v)
        _gather_start(nids_hbm, idx_v, cid_v, sem_n).wait()
        for k in range(B * 8 // L):
            v = plsc.load_gather(cid_v, [pat_q + 2 * k, pat_j])
            cix_v[k // 8, pl.ds((k % 8) * L, L)] = v
        for c in range(nch):
            _gather_start(data_hbm, cix_v.at[c],
                          rows_v.at[pl.ds(c * 128, 128), :], sem_d)

    def stage2(qbase, rows_v):
        for j in range(8):
            pltpu.sync_copy(wt_hbm.at[j, pl.ds(qbase, B)], w_v.at[j])
        _drain(data_hbm, rows_v, sem_d)

        def qstep(p, c):
            w_pair = plsc.load_gather(w_v, [pat_j, pat_q + 2 * p])
            for h in range(2):
                q = 2 * p + h
                acc0 = jnp.zeros((L,), jnp.float32)
                acc1 = jnp.zeros((L,), jnp.float32)
                for j in range(8):
                    wj = w_pair.at[jnp.full((L,), 8 * h + j, jnp.int32)].get(
                        mode="promise_in_bounds")
                    r0 = rows_v[8 * q + j, pl.ds(0, L)]
                    r1 = rows_v[8 * q + j, pl.ds(D - L, L)]
                    acc0 = acc0 + wj * r0
                    acc1 = acc1 + wj * r1
                out_v[q, pl.ds(0, L)] = acc0
                out_v[q, pl.ds(D - L, L)] = acc1
            return c

        lax.fori_loop(0, B // 2, qstep, 0)
        pltpu.sync_copy(out_v, out_hbm.at[pl.ds(qbase, B)])

    stage1(wstart, cix0, rows0)

    def pair(t, carry):
        g0 = 2 * t
        qb0 = wstart + g0 * B
        stage1(qb0 + B, cix1, rows1)
        stage2(qb0, rows0)

        @pl.when(g0 + 2 < nblk)
        def _():
            stage1(qb0 + 2 * B, cix0, rows0)

        stage2(qb0 + B, rows1)
        return carry

    lax.fori_loop(0, nblk // 2, pair, 0)


def kernel(indices, nids, data, weights):
    n = indices.shape[0]
    nnodes = nids.shape[0]
    mesh = plsc.VectorSubcoreMesh(core_axis_name="c", subcore_axis_name="s",
                                  num_cores=NC, num_subcores=NS)
    data32 = jnp.pad(data, ((0, 0), (0, 32 - D)))
    tr_f = pl.kernel(
        _transpose_body,
        out_type=jax.ShapeDtypeStruct((nnodes, 8), jnp.int32),
        mesh=mesh,
        compiler_params=_params,
        scratch_types=[
            pltpu.VMEM((8, CT), jnp.int32),
            pltpu.VMEM((CT, 8), jnp.int32),
        ],
    )
    nids_rm = tr_f(nids.T)
    f = pl.kernel(
        _body,
        out_type=jax.ShapeDtypeStruct((n, D), jnp.float32),
        mesh=mesh,
        compiler_params=_params,
        scratch_types=[
            pltpu.VMEM((B,), jnp.int32),
            pltpu.VMEM((B, 8), jnp.int32),
            pltpu.VMEM((B * 8 // 128, 128), jnp.int32),
            pltpu.VMEM((B * 8 // 128, 128), jnp.int32),
            pltpu.VMEM((8, B), jnp.float32),
            pltpu.VMEM((B * 8, 32), jnp.float32),
            pltpu.VMEM((B * 8, 32), jnp.float32),
            pltpu.VMEM((B, D), jnp.float32),
            pltpu.SemaphoreType.DMA,
            pltpu.SemaphoreType.DMA,
        ],
    )
    return f(indices, nids_rm, data32, weights.T)


# speedup vs baseline: 1.3983x; 1.2185x over previous
import jax
import jax.numpy as jnp
from jax import lax
from jax.experimental import pallas as pl
from jax.experimental.pallas import tpu as pltpu
from jax.experimental.pallas import tpu_sc as plsc

D = 28
L = 16
NC, NS = 2, 16
NW = NC * NS
B = 128
CT = 2048

_params = pltpu.CompilerParams(use_tc_tiling_on_sc=False,
                               needs_layout_passes=False)


def _worker_id():
    return lax.axis_index("s") * NC + lax.axis_index("c")


def _gather_start(src_hbm, idx_ref, dst, sem):
    return pltpu.async_copy(src_hbm.at[idx_ref], dst, sem)


def _transpose_body(nt_hbm, out_hbm, in_v, out_v):
    ncols = nt_hbm.shape[1]
    cpw = ncols // NW
    cstart = _worker_id() * cpw
    io = lax.iota(jnp.int32, L)

    def blk(g, carry):
        base = cstart + g * CT
        pltpu.sync_copy(nt_hbm.at[:, pl.ds(base, CT)], in_v)

        def grp(t, c):
            o = t * L
            for j in range(8):
                v = in_v[j, pl.ds(o, L)]
                plsc.store_scatter(out_v, [o + io, jnp.full((L,), j, jnp.int32)], v)
            return c

        lax.fori_loop(0, CT // L, grp, 0)
        pltpu.sync_copy(out_v, out_hbm.at[pl.ds(base, CT)])
        return carry

    lax.fori_loop(0, cpw // CT, blk, 0)


def _drain(data_hbm, rows_v, sem):
    pltpu.make_async_copy(data_hbm.at[pl.ds(0, rows_v.shape[0])],
                          rows_v, sem).wait()


def _body(idx_hbm, nids_hbm, data_hbm, wt_hbm, out_hbm,
          idx_v, cid_v, cix0, cix1, w_v, rows0, rows1, out_v, sem_n, sem_d):
    n = idx_hbm.shape[0]
    qpw = n // NW
    nblk = qpw // B
    wstart = _worker_id() * qpw
    nch = (B * 8) // 128
    io = lax.iota(jnp.int32, L)
    pat_q = lax.shift_right_logical(io, 3)
    pat_j = lax.bitwise_and(io, 7)

    def stage1(qbase, cix_v, rows_v):
        pltpu.sync_copy(idx_hbm.at[pl.ds(qbase, B)], idx_v)
        _gather_start(nids_hbm, idx_v, cid_v, sem_n).wait()
        for k in range(B * 8 // L):
            v = plsc.load_gather(cid_v, [pat_q + 2 * k, pat_j])
            cix_v[k // 8, pl.ds((k % 8) * L, L)] = v
        for c in range(nch):
            _gather_start(data_hbm, cix_v.at[c],
                          rows_v.at[pl.ds(c * 128, 128), :], sem_d)

    def stage2(qbase, rows_v):
        pltpu.sync_copy(wt_hbm.at[:, pl.ds(qbase, B)], w_v)
        _drain(data_hbm, rows_v, sem_d)

        def qstep(p, c):
            w_pair = plsc.load_gather(w_v, [pat_j, pat_q + 2 * p])
            for h in range(2):
                q = 2 * p + h
                acc0 = jnp.zeros((L,), jnp.float32)
                acc1 = jnp.zeros((L,), jnp.float32)
                for j in range(8):
                    wj = w_pair.at[jnp.full((L,), 8 * h + j, jnp.int32)].get(
                        mode="promise_in_bounds")
                    r0 = rows_v[8 * q + j, pl.ds(0, L)]
                    r1 = rows_v[8 * q + j, pl.ds(D - L, L)]
                    acc0 = acc0 + wj * r0
                    acc1 = acc1 + wj * r1
                out_v[q, pl.ds(0, L)] = acc0
                out_v[q, pl.ds(D - L, L)] = acc1
            return c

        lax.fori_loop(0, B // 2, qstep, 0)
        pltpu.sync_copy(out_v, out_hbm.at[pl.ds(qbase, B)])

    stage1(wstart, cix0, rows0)

    def pair(t, carry):
        g0 = 2 * t
        qb0 = wstart + g0 * B
        stage1(qb0 + B, cix1, rows1)
        stage2(qb0, rows0)

        @pl.when(g0 + 2 < nblk)
        def _():
            stage1(qb0 + 2 * B, cix0, rows0)

        stage2(qb0 + B, rows1)
        return carry

    lax.fori_loop(0, nblk // 2, pair, 0)


def kernel(indices, nids, data, weights):
    n = indices.shape[0]
    nnodes = nids.shape[0]
    mesh = plsc.VectorSubcoreMesh(core_axis_name="c", subcore_axis_name="s",
                                  num_cores=NC, num_subcores=NS)
    data32 = jnp.pad(data, ((0, 0), (0, 32 - D)))
    tr_f = pl.kernel(
        _transpose_body,
        out_type=jax.ShapeDtypeStruct((nnodes, 8), jnp.int32),
        mesh=mesh,
        compiler_params=_params,
        scratch_types=[
            pltpu.VMEM((8, CT), jnp.int32),
            pltpu.VMEM((CT, 8), jnp.int32),
        ],
    )
    nids_rm = tr_f(nids.T)
    f = pl.kernel(
        _body,
        out_type=jax.ShapeDtypeStruct((n, D), jnp.float32),
        mesh=mesh,
        compiler_params=_params,
        scratch_types=[
            pltpu.VMEM((B,), jnp.int32),
            pltpu.VMEM((B, 8), jnp.int32),
            pltpu.VMEM((B * 8 // 128, 128), jnp.int32),
            pltpu.VMEM((B * 8 // 128, 128), jnp.int32),
            pltpu.VMEM((8, B), jnp.float32),
            pltpu.VMEM((B * 8, 32), jnp.float32),
            pltpu.VMEM((B * 8, 32), jnp.float32),
            pltpu.VMEM((B, D), jnp.float32),
            pltpu.SemaphoreType.DMA,
            pltpu.SemaphoreType.DMA,
        ],
    )
    return f(indices, nids_rm, data32, weights.T)
